# Initial kernel scaffold; baseline (speedup 1.0000x reference)
import jax
import jax.numpy as jnp
from jax.experimental import pallas as pl


def kernel(x, edge_index, batch, g0_W1, g0_W2, g0_gamma, g0_beta, g1_W1, g1_W2, g1_gamma, g1_beta, fc1_W, fc1_b, fc2_W, fc2_b):
    raise NotImplementedError("write your pallas kernel here")


# speedup vs baseline: 2.9783x; 2.9783x over previous
import functools

import jax
import jax.numpy as jnp
from jax import lax
from jax.experimental import pallas as pl
from jax.experimental.pallas import tpu as pltpu
from jax.experimental.pallas import tpu_sc as plsc

_N = 10000
_D = 128
_E = 320000
_NC, _NS = 2, 16
_NW = _NC * _NS
_K = 128
_CH = 80
_EW = _K * _CH
_EPAD = _NW * _EW
_NPAD = 10240
_RPW = _NPAD // _NS

def _sc_agg_body(x_hbm, srcs_hbm, dsts_hbm, out_hbm, src_v, dst_v, rowbuf, agg_sh, gsem):
    c = lax.axis_index("c")
    s = lax.axis_index("s")
    wid = s * _NC + c

    def _zrow(i, carry):
        for j in range(_D // 16):
            rowbuf[i, 16 * j:16 * (j + 1)] = jnp.zeros((16,), jnp.float32)
        return carry
    lax.fori_loop(0, _K, _zrow, 0)
    for t in range(_RPW // _K):
        pltpu.sync_copy(rowbuf, agg_sh.at[pl.ds(s * _RPW + t * _K, _K)])
    plsc.subcore_barrier()

    pltpu.sync_copy(srcs_hbm.at[wid], src_v)
    pltpu.sync_copy(dsts_hbm.at[wid], dst_v)

    def _edge_chunk(j, carry):
        pltpu.async_copy(x_hbm.at[src_v.at[j]], rowbuf, gsem).wait()
        pltpu.sync_copy(rowbuf, agg_sh.at[dst_v.at[j]], add=True)
        return carry
    lax.fori_loop(0, _CH, _edge_chunk, 0)

    plsc.subcore_barrier()
    pltpu.sync_copy(
        agg_sh.at[pl.ds(s * _RPW, _RPW)],
        out_hbm.at[c, pl.ds(s * _RPW, _RPW)],
    )


@functools.cache
def _sc_agg_call():
    mesh = plsc.VectorSubcoreMesh(
        core_axis_name="c", subcore_axis_name="s",
        num_cores=_NC, num_subcores=_NS)
    return pl.kernel(
        _sc_agg_body,
        out_type=jax.ShapeDtypeStruct((_NC, _NPAD, _D), jnp.float32),
        mesh=mesh,
        scratch_types=[
            pltpu.VMEM((_CH, _K), jnp.int32),
            pltpu.VMEM((_CH, _K), jnp.int32),
            pltpu.VMEM((_K, _D), jnp.float32),
            pltpu.VMEM_SHARED((_NPAD, _D), jnp.float32),
            pltpu.SemaphoreType.DMA,
        ],
    )


def _sc_agg(x, srcs, dsts):
    return _sc_agg_call()(x, srcs, dsts)


def _gin_body(x_ref, pa_ref, pb_ref, w1_ref, w2_ref, gm_ref, bt_ref, o_ref, *, final_relu):
    h = x_ref[...] + pa_ref[...] + pb_ref[...]
    a = jnp.maximum(jnp.dot(h, w1_ref[...], preferred_element_type=jnp.float32), 0.0)
    b = jnp.maximum(jnp.dot(a, w2_ref[...], preferred_element_type=jnp.float32), 0.0)
    m = jnp.mean(b, axis=0, keepdims=True)
    v = jnp.mean(jnp.square(b - m), axis=0, keepdims=True)
    h2 = (b - m) / jnp.sqrt(v + 1e-5) * gm_ref[...] + bt_ref[...]
    o_ref[...] = jnp.maximum(h2, 0.0) if final_relu else h2


def _l1_head_body(x_ref, pa_ref, pb_ref, w1_ref, w2_ref, gm_ref, bt_ref,
                  f1w_ref, f1b_ref, f2w_ref, f2b_ref, u_ref, nz_ref, o_ref):
    h = x_ref[...] + pa_ref[...] + pb_ref[...]
    a = jnp.maximum(jnp.dot(h, w1_ref[...], preferred_element_type=jnp.float32), 0.0)
    b = jnp.maximum(jnp.dot(a, w2_ref[...], preferred_element_type=jnp.float32), 0.0)
    m = jnp.mean(b, axis=0, keepdims=True)
    v = jnp.mean(jnp.square(b - m), axis=0, keepdims=True)
    nf = (b - m) / jnp.sqrt(v + 1e-5) * gm_ref[...] + bt_ref[...]

    a1 = jnp.tanh(jnp.dot(nf, f1w_ref[...], preferred_element_type=jnp.float32)
                  + f1b_ref[...])
    logits = jnp.dot(a1, f2w_ref[...], preferred_element_type=jnp.float32) + f2b_ref[...]
    col = lax.broadcasted_iota(jnp.int32, (1, _D), 1)
    mask = col < 2
    z = jnp.where(mask, logits, -1e30)
    zm = jnp.max(z, axis=1, keepdims=True)
    e = jnp.where(mask, jnp.exp(z - zm), 0.0)
    assign = e / jnp.sum(e, axis=1, keepdims=True)

    gnoise = -jnp.log(-jnp.log(u_ref[...]))
    g = assign + gnoise
    z2 = jnp.where(mask, g, -1e30)
    zm2 = jnp.max(z2, axis=1, keepdims=True)
    e2 = jnp.where(mask, jnp.exp(z2 - zm2), 0.0)
    gum = e2 / jnp.sum(e2, axis=1, keepdims=True)
    lam_p = gum[:, 0:1]
    lam_n = gum[:, 1:2]

    mu = jnp.mean(nf, axis=0, keepdims=True)
    sd = jnp.sqrt(jnp.sum(jnp.square(nf - mu), axis=0, keepdims=True) / (_N - 1))
    o_ref[...] = lam_p * nf + lam_n * mu + nz_ref[...] * (lam_n * sd)


_gin_call = {
    fr: pl.pallas_call(
        functools.partial(_gin_body, final_relu=fr),
        out_shape=jax.ShapeDtypeStruct((_N, _D), jnp.float32),
    )
    for fr in (True,)
}

_l1_head_call = pl.pallas_call(
    _l1_head_body,
    out_shape=jax.ShapeDtypeStruct((_N, _D), jnp.float32),
)


def kernel(x, edge_index, batch, g0_W1, g0_W2, g0_gamma, g0_beta,
           g1_W1, g1_W2, g1_gamma, g1_beta, fc1_W, fc1_b, fc2_W, fc2_b):
    src = edge_index[0]
    dst = edge_index[1]
    pad = _EPAD - _E
    srcs = jnp.concatenate([src, jnp.zeros((pad,), jnp.int32)]).reshape(_NW, _CH, _K)
    dsts = jnp.concatenate([dst, jnp.full((pad,), _N, jnp.int32)]).reshape(_NW, _CH, _K)

    parts0 = _sc_agg(x, srcs, dsts)
    nf1 = _gin_call[True](
        x, parts0[0, :_N], parts0[1, :_N], g0_W1, g0_W2,
        g0_gamma.reshape(1, _D), g0_beta.reshape(1, _D))

    parts1 = _sc_agg(nf1, srcs, dsts)

    u = jax.random.uniform(jax.random.key(42), (_N, 2), minval=1e-10, maxval=1.0)
    u_pad = jnp.concatenate(
        [u, jnp.full((_N, _D - 2), 0.5, jnp.float32)], axis=1)
    nz = jax.random.uniform(jax.random.key(7), (_N, _D), dtype=jnp.float32)
    f2w_pad = jnp.zeros((_D, _D), jnp.float32).at[:, :2].set(fc2_W)
    f2b_pad = jnp.zeros((1, _D), jnp.float32).at[0, :2].set(fc2_b)

    out = _l1_head_call(
        nf1, parts1[0, :_N], parts1[1, :_N], g1_W1, g1_W2,
        g1_gamma.reshape(1, _D), g1_beta.reshape(1, _D),
        fc1_W, fc1_b.reshape(1, _D), f2w_pad, f2b_pad, u_pad, nz)
    return out


# speedup vs baseline: 3.3536x; 1.1260x over previous
import functools

import jax
import jax.numpy as jnp
from jax import lax
from jax.experimental import pallas as pl
from jax.experimental.pallas import tpu as pltpu
from jax.experimental.pallas import tpu_sc as plsc

_N = 10000
_D = 128
_E = 320000
_NC, _NS = 2, 16
_NW = _NC * _NS
_K = 128
_CH = 80
_EW = _K * _CH
_EPAD = _NW * _EW
_NPAD = 10240
_RPW = _NPAD // _NS

_NBUF = 2
_CHH = _CH // 2


def _sc_agg_body(x_hbm, srcs_hbm, dsts_hbm, out_hbm, src_v, dst_v, rowbuf, agg_sh, gsem, ssem):
    c = lax.axis_index("c")
    s = lax.axis_index("s")
    wid = s * _NC + c

    def _zrow(i, carry):
        for j in range(_D // 16):
            rowbuf[0, i, 16 * j:16 * (j + 1)] = jnp.zeros((16,), jnp.float32)
        return carry
    lax.fori_loop(0, _K, _zrow, 0)
    for t in range(_RPW // _K):
        pltpu.sync_copy(rowbuf.at[0], agg_sh.at[pl.ds(s * _RPW + t * _K, _K)])
    plsc.subcore_barrier()

    for h in range(2):
        pltpu.sync_copy(srcs_hbm.at[wid, pl.ds(h * _CHH, _CHH)], src_v)
        pltpu.sync_copy(dsts_hbm.at[wid, pl.ds(h * _CHH, _CHH)], dst_v)
        pltpu.async_copy(x_hbm.at[src_v.at[0]], rowbuf.at[0], gsem)

        def _edge_pair(g, carry):
            for b in range(_NBUF):
                j = g * _NBUF + b
                bo = 1 - b

                @pl.when(j >= 1)
                def _wait_scatter():
                    pltpu.make_async_copy(
                        rowbuf.at[bo], agg_sh.at[dst_v.at[j - 1]], ssem).wait()

                @pl.when(j + 1 < _CHH)
                def _issue_gather():
                    pltpu.async_copy(x_hbm.at[src_v.at[j + 1]], rowbuf.at[bo], gsem)

                pltpu.make_async_copy(x_hbm.at[src_v.at[j]], rowbuf.at[b], gsem).wait()
                pltpu.async_copy(rowbuf.at[b], agg_sh.at[dst_v.at[j]], ssem, add=True)
            return carry
        lax.fori_loop(0, _CHH // _NBUF, _edge_pair, 0)
        pltpu.make_async_copy(
            rowbuf.at[(_CHH - 1) % _NBUF], agg_sh.at[dst_v.at[_CHH - 1]], ssem).wait()

    plsc.subcore_barrier()
    pltpu.sync_copy(
        agg_sh.at[pl.ds(s * _RPW, _RPW)],
        out_hbm.at[c, pl.ds(s * _RPW, _RPW)],
    )


@functools.cache
def _sc_agg_call():
    mesh = plsc.VectorSubcoreMesh(
        core_axis_name="c", subcore_axis_name="s",
        num_cores=_NC, num_subcores=_NS)
    return pl.kernel(
        _sc_agg_body,
        out_type=jax.ShapeDtypeStruct((_NC, _NPAD, _D), jnp.float32),
        mesh=mesh,
        scratch_types=[
            pltpu.VMEM((_CHH, _K), jnp.int32),
            pltpu.VMEM((_CHH, _K), jnp.int32),
            pltpu.VMEM((_NBUF, _K, _D), jnp.float32),
            pltpu.VMEM_SHARED((_NPAD, _D), jnp.float32),
            pltpu.SemaphoreType.DMA,
            pltpu.SemaphoreType.DMA,
        ],
    )


def _sc_agg(x, srcs, dsts):
    return _sc_agg_call()(x, srcs, dsts)


def _gin_body(x_ref, pa_ref, pb_ref, w1_ref, w2_ref, gm_ref, bt_ref, o_ref, *, final_relu):
    h = x_ref[...] + pa_ref[...] + pb_ref[...]
    a = jnp.maximum(jnp.dot(h, w1_ref[...], preferred_element_type=jnp.float32), 0.0)
    b = jnp.maximum(jnp.dot(a, w2_ref[...], preferred_element_type=jnp.float32), 0.0)
    m = jnp.mean(b, axis=0, keepdims=True)
    v = jnp.mean(jnp.square(b - m), axis=0, keepdims=True)
    h2 = (b - m) / jnp.sqrt(v + 1e-5) * gm_ref[...] + bt_ref[...]
    o_ref[...] = jnp.maximum(h2, 0.0) if final_relu else h2


def _l1_head_body(x_ref, pa_ref, pb_ref, w1_ref, w2_ref, gm_ref, bt_ref,
                  f1w_ref, f1b_ref, f2w_ref, f2b_ref, u_ref, nz_ref, o_ref):
    h = x_ref[...] + pa_ref[...] + pb_ref[...]
    a = jnp.maximum(jnp.dot(h, w1_ref[...], preferred_element_type=jnp.float32), 0.0)
    b = jnp.maximum(jnp.dot(a, w2_ref[...], preferred_element_type=jnp.float32), 0.0)
    m = jnp.mean(b, axis=0, keepdims=True)
    v = jnp.mean(jnp.square(b - m), axis=0, keepdims=True)
    nf = (b - m) / jnp.sqrt(v + 1e-5) * gm_ref[...] + bt_ref[...]

    a1 = jnp.tanh(jnp.dot(nf, f1w_ref[...], preferred_element_type=jnp.float32)
                  + f1b_ref[...])
    logits = jnp.dot(a1, f2w_ref[...], preferred_element_type=jnp.float32) + f2b_ref[...]
    col = lax.broadcasted_iota(jnp.int32, (1, _D), 1)
    mask = col < 2
    z = jnp.where(mask, logits, -1e30)
    zm = jnp.max(z, axis=1, keepdims=True)
    e = jnp.where(mask, jnp.exp(z - zm), 0.0)
    assign = e / jnp.sum(e, axis=1, keepdims=True)

    gnoise = -jnp.log(-jnp.log(u_ref[...]))
    g = assign + gnoise
    z2 = jnp.where(mask, g, -1e30)
    zm2 = jnp.max(z2, axis=1, keepdims=True)
    e2 = jnp.where(mask, jnp.exp(z2 - zm2), 0.0)
    gum = e2 / jnp.sum(e2, axis=1, keepdims=True)
    lam_p = gum[:, 0:1]
    lam_n = gum[:, 1:2]

    mu = jnp.mean(nf, axis=0, keepdims=True)
    sd = jnp.sqrt(jnp.sum(jnp.square(nf - mu), axis=0, keepdims=True) / (_N - 1))
    o_ref[...] = lam_p * nf + lam_n * mu + nz_ref[...] * (lam_n * sd)


_gin_call = {
    fr: pl.pallas_call(
        functools.partial(_gin_body, final_relu=fr),
        out_shape=jax.ShapeDtypeStruct((_N, _D), jnp.float32),
    )
    for fr in (True,)
}

_l1_head_call = pl.pallas_call(
    _l1_head_body,
    out_shape=jax.ShapeDtypeStruct((_N, _D), jnp.float32),
)


def kernel(x, edge_index, batch, g0_W1, g0_W2, g0_gamma, g0_beta,
           g1_W1, g1_W2, g1_gamma, g1_beta, fc1_W, fc1_b, fc2_W, fc2_b):
    src = edge_index[0]
    dst = edge_index[1]
    pad = _EPAD - _E
    srcs = jnp.concatenate([src, jnp.zeros((pad,), jnp.int32)]).reshape(_NW, _CH, _K)
    dsts = jnp.concatenate([dst, jnp.full((pad,), _N, jnp.int32)]).reshape(_NW, _CH, _K)

    parts0 = _sc_agg(x, srcs, dsts)
    nf1 = _gin_call[True](
        x, parts0[0, :_N], parts0[1, :_N], g0_W1, g0_W2,
        g0_gamma.reshape(1, _D), g0_beta.reshape(1, _D))

    parts1 = _sc_agg(nf1, srcs, dsts)

    u = jax.random.uniform(jax.random.key(42), (_N, 2), minval=1e-10, maxval=1.0)
    u_pad = jnp.concatenate(
        [u, jnp.full((_N, _D - 2), 0.5, jnp.float32)], axis=1)
    nz = jax.random.uniform(jax.random.key(7), (_N, _D), dtype=jnp.float32)
    f2w_pad = jnp.zeros((_D, _D), jnp.float32).at[:, :2].set(fc2_W)
    f2b_pad = jnp.zeros((1, _D), jnp.float32).at[0, :2].set(fc2_b)

    out = _l1_head_call(
        nf1, parts1[0, :_N], parts1[1, :_N], g1_W1, g1_W2,
        g1_gamma.reshape(1, _D), g1_beta.reshape(1, _D),
        fc1_W, fc1_b.reshape(1, _D), f2w_pad, f2b_pad, u_pad, nz)
    return out


# speedup vs baseline: 11.3320x; 3.3791x over previous
import functools

import jax
import jax.numpy as jnp
from jax import lax
from jax.experimental import pallas as pl
from jax.experimental.pallas import tpu as pltpu
from jax.experimental.pallas import tpu_sc as plsc

_N = 10000
_D = 128
_E = 320000
_NC, _NS = 2, 16
_NW = _NC * _NS
_K = 64
_CH = 160
_EW = _K * _CH
_EPAD = _NW * _EW
_NPAD = 10240
_RPW = _NPAD // _NS

_NBUF = 4
_CHH = _CH // 4


def _sc_agg_body(x_hbm, srcs_hbm, dsts_hbm, out_hbm, src_v, dst_v, rowbuf, agg_sh, gsem, ssem):
    c = lax.axis_index("c")
    s = lax.axis_index("s")
    wid = s * _NC + c

    def _zrow(i, carry):
        for j in range(_D // 16):
            rowbuf[0, i, 16 * j:16 * (j + 1)] = jnp.zeros((16,), jnp.float32)
        return carry
    lax.fori_loop(0, _K, _zrow, 0)
    for t in range(_RPW // _K):
        pltpu.sync_copy(rowbuf.at[0], agg_sh.at[pl.ds(s * _RPW + t * _K, _K)])
    plsc.subcore_barrier()

    for h in range(4):
        pltpu.sync_copy(srcs_hbm.at[wid, pl.ds(h * _CHH, _CHH)], src_v)
        pltpu.sync_copy(dsts_hbm.at[wid, pl.ds(h * _CHH, _CHH)], dst_v)
        for b in range(2):
            pltpu.async_copy(x_hbm.at[src_v.at[b]], rowbuf.at[b], gsem)

        def _edge_group(g, carry):
            for b in range(_NBUF):
                j = g * _NBUF + b
                bo = (b + 2) % _NBUF

                @pl.when(j >= 2)
                def _wait_scatter():
                    pltpu.make_async_copy(
                        rowbuf.at[bo], agg_sh.at[dst_v.at[j - 2]], ssem).wait()

                @pl.when(j + 2 < _CHH)
                def _issue_gather():
                    pltpu.async_copy(x_hbm.at[src_v.at[j + 2]], rowbuf.at[bo], gsem)

                pltpu.make_async_copy(x_hbm.at[src_v.at[j]], rowbuf.at[b], gsem).wait()
                pltpu.async_copy(rowbuf.at[b], agg_sh.at[dst_v.at[j]], ssem, add=True)
            return carry
        lax.fori_loop(0, _CHH // _NBUF, _edge_group, 0)
        for j in (_CHH - 2, _CHH - 1):
            pltpu.make_async_copy(
                rowbuf.at[j % _NBUF], agg_sh.at[dst_v.at[j]], ssem).wait()

    plsc.subcore_barrier()
    pltpu.sync_copy(
        agg_sh.at[pl.ds(s * _RPW, _RPW)],
        out_hbm.at[c, pl.ds(s * _RPW, _RPW)],
    )


@functools.cache
def _sc_agg_call():
    mesh = plsc.VectorSubcoreMesh(
        core_axis_name="c", subcore_axis_name="s",
        num_cores=_NC, num_subcores=_NS)
    return pl.kernel(
        _sc_agg_body,
        out_type=jax.ShapeDtypeStruct((_NC, _NPAD, _D), jnp.float32),
        mesh=mesh,
        scratch_types=[
            pltpu.VMEM((_CHH, _K), jnp.int32),
            pltpu.VMEM((_CHH, _K), jnp.int32),
            pltpu.VMEM((_NBUF, _K, _D), jnp.float32),
            pltpu.VMEM_SHARED((_NPAD, _D), jnp.float32),
            pltpu.SemaphoreType.DMA,
            pltpu.SemaphoreType.DMA,
        ],
    )


def _sc_agg(x, srcs, dsts):
    return _sc_agg_call()(x, srcs, dsts)


def _gin_body(x_ref, pa_ref, pb_ref, w1_ref, w2_ref, gm_ref, bt_ref, o_ref, *, final_relu):
    h = x_ref[...] + pa_ref[...] + pb_ref[...]
    a = jnp.maximum(jnp.dot(h, w1_ref[...], preferred_element_type=jnp.float32), 0.0)
    b = jnp.maximum(jnp.dot(a, w2_ref[...], preferred_element_type=jnp.float32), 0.0)
    m = jnp.mean(b, axis=0, keepdims=True)
    v = jnp.mean(jnp.square(b - m), axis=0, keepdims=True)
    h2 = (b - m) / jnp.sqrt(v + 1e-5) * gm_ref[...] + bt_ref[...]
    o_ref[...] = jnp.maximum(h2, 0.0) if final_relu else h2


def _l1_head_body(x_ref, pa_ref, pb_ref, w1_ref, w2_ref, gm_ref, bt_ref,
                  f1w_ref, f1b_ref, f2w_ref, f2b_ref, u_ref, nz_ref, o_ref):
    h = x_ref[...] + pa_ref[...] + pb_ref[...]
    a = jnp.maximum(jnp.dot(h, w1_ref[...], preferred_element_type=jnp.float32), 0.0)
    b = jnp.maximum(jnp.dot(a, w2_ref[...], preferred_element_type=jnp.float32), 0.0)
    m = jnp.mean(b, axis=0, keepdims=True)
    v = jnp.mean(jnp.square(b - m), axis=0, keepdims=True)
    nf = (b - m) / jnp.sqrt(v + 1e-5) * gm_ref[...] + bt_ref[...]

    a1 = jnp.tanh(jnp.dot(nf, f1w_ref[...], preferred_element_type=jnp.float32)
                  + f1b_ref[...])
    logits = jnp.dot(a1, f2w_ref[...], preferred_element_type=jnp.float32) + f2b_ref[...]
    col = lax.broadcasted_iota(jnp.int32, (1, _D), 1)
    mask = col < 2
    z = jnp.where(mask, logits, -1e30)
    zm = jnp.max(z, axis=1, keepdims=True)
    e = jnp.where(mask, jnp.exp(z - zm), 0.0)
    assign = e / jnp.sum(e, axis=1, keepdims=True)

    gnoise = -jnp.log(-jnp.log(u_ref[...]))
    g = assign + gnoise
    z2 = jnp.where(mask, g, -1e30)
    zm2 = jnp.max(z2, axis=1, keepdims=True)
    e2 = jnp.where(mask, jnp.exp(z2 - zm2), 0.0)
    gum = e2 / jnp.sum(e2, axis=1, keepdims=True)
    lam_p = gum[:, 0:1]
    lam_n = gum[:, 1:2]

    mu = jnp.mean(nf, axis=0, keepdims=True)
    sd = jnp.sqrt(jnp.sum(jnp.square(nf - mu), axis=0, keepdims=True) / (_N - 1))
    o_ref[...] = lam_p * nf + lam_n * mu + nz_ref[...] * (lam_n * sd)


_gin_call = {
    fr: pl.pallas_call(
        functools.partial(_gin_body, final_relu=fr),
        out_shape=jax.ShapeDtypeStruct((_N, _D), jnp.float32),
    )
    for fr in (True,)
}

_l1_head_call = pl.pallas_call(
    _l1_head_body,
    out_shape=jax.ShapeDtypeStruct((_N, _D), jnp.float32),
)


def kernel(x, edge_index, batch, g0_W1, g0_W2, g0_gamma, g0_beta,
           g1_W1, g1_W2, g1_gamma, g1_beta, fc1_W, fc1_b, fc2_W, fc2_b):
    src = edge_index[0]
    dst = edge_index[1]
    pad = _EPAD - _E
    pad_src = (jnp.arange(pad, dtype=jnp.int32) * 37) % _N
    pad_dst = _N + (jnp.arange(pad, dtype=jnp.int32) % (_NPAD - _N))
    srcs = jnp.concatenate([src, pad_src]).reshape(_NW, _CH, _K)
    dsts = jnp.concatenate([dst, pad_dst]).reshape(_NW, _CH, _K)

    parts0 = _sc_agg(x, srcs, dsts)
    nf1 = _gin_call[True](
        x, parts0[0, :_N], parts0[1, :_N], g0_W1, g0_W2,
        g0_gamma.reshape(1, _D), g0_beta.reshape(1, _D))

    parts1 = _sc_agg(nf1, srcs, dsts)

    u = jax.random.uniform(jax.random.key(42), (_N, 2), minval=1e-10, maxval=1.0)
    u_pad = jnp.concatenate(
        [u, jnp.full((_N, _D - 2), 0.5, jnp.float32)], axis=1)
    nz = jax.random.uniform(jax.random.key(7), (_N, _D), dtype=jnp.float32)
    f2w_pad = jnp.zeros((_D, _D), jnp.float32).at[:, :2].set(fc2_W)
    f2b_pad = jnp.zeros((1, _D), jnp.float32).at[0, :2].set(fc2_b)

    out = _l1_head_call(
        nf1, parts1[0, :_N], parts1[1, :_N], g1_W1, g1_W2,
        g1_gamma.reshape(1, _D), g1_beta.reshape(1, _D),
        fc1_W, fc1_b.reshape(1, _D), f2w_pad, f2b_pad, u_pad, nz)
    return out


# speedup vs baseline: 11.9297x; 1.0527x over previous
import functools

import jax
import jax.numpy as jnp
from jax import lax
from jax.experimental import pallas as pl
from jax.experimental.pallas import tpu as pltpu
from jax.experimental.pallas import tpu_sc as plsc

_N = 10000
_D = 128
_E = 320000
_NC, _NS = 2, 16
_NW = _NC * _NS
_K = 64
_CH = 160
_EW = _K * _CH
_EPAD = _NW * _EW
_NPAD = 10240
_RPW = _NPAD // _NS

_NBUF = 4
_CHH = _CH // 4


def _sc_agg_body(x_hbm, srcs_hbm, dsts_hbm, out_hbm, src_v, dst_v, rowbuf, agg_sh, gsem, ssem):
    c = lax.axis_index("c")
    s = lax.axis_index("s")
    wid = s * _NC + c

    def _zrow(i, carry):
        for j in range(_D // 16):
            rowbuf[0, i, 16 * j:16 * (j + 1)] = jnp.zeros((16,), jnp.float32)
        return carry
    lax.fori_loop(0, _K, _zrow, 0)
    for t in range(_RPW // _K):
        pltpu.sync_copy(rowbuf.at[0], agg_sh.at[pl.ds(s * _RPW + t * _K, _K)])
    plsc.subcore_barrier()

    for h in range(4):
        pltpu.sync_copy(srcs_hbm.at[wid, pl.ds(h * _CHH, _CHH)], src_v)
        pltpu.sync_copy(dsts_hbm.at[wid, pl.ds(h * _CHH, _CHH)], dst_v)
        for b in range(2):
            pltpu.async_copy(x_hbm.at[src_v.at[b]], rowbuf.at[b], gsem)

        def _edge_group(g, carry):
            for b in range(_NBUF):
                j = g * _NBUF + b
                bo = (b + 2) % _NBUF

                @pl.when(j >= 2)
                def _wait_scatter():
                    pltpu.make_async_copy(
                        rowbuf.at[bo], agg_sh.at[dst_v.at[j - 2]], ssem).wait()

                @pl.when(j + 2 < _CHH)
                def _issue_gather():
                    pltpu.async_copy(x_hbm.at[src_v.at[j + 2]], rowbuf.at[bo], gsem)

                pltpu.make_async_copy(x_hbm.at[src_v.at[j]], rowbuf.at[b], gsem).wait()
                pltpu.async_copy(rowbuf.at[b], agg_sh.at[dst_v.at[j]], ssem, add=True)
            return carry
        lax.fori_loop(0, _CHH // _NBUF, _edge_group, 0)
        for j in (_CHH - 2, _CHH - 1):
            pltpu.make_async_copy(
                rowbuf.at[j % _NBUF], agg_sh.at[dst_v.at[j]], ssem).wait()

    plsc.subcore_barrier()
    pltpu.sync_copy(
        agg_sh.at[pl.ds(s * _RPW, _RPW)],
        out_hbm.at[c, pl.ds(s * _RPW, _RPW)],
    )


@functools.cache
def _sc_agg_call():
    mesh = plsc.VectorSubcoreMesh(
        core_axis_name="c", subcore_axis_name="s",
        num_cores=_NC, num_subcores=_NS)
    return pl.kernel(
        _sc_agg_body,
        out_type=jax.ShapeDtypeStruct((_NC, _NPAD, _D), jnp.float32),
        mesh=mesh,
        scratch_types=[
            pltpu.VMEM((_CHH, _K), jnp.int32),
            pltpu.VMEM((_CHH, _K), jnp.int32),
            pltpu.VMEM((_NBUF, _K, _D), jnp.float32),
            pltpu.VMEM_SHARED((_NPAD, _D), jnp.float32),
            pltpu.SemaphoreType.DMA,
            pltpu.SemaphoreType.DMA,
        ],
    )


def _sc_agg(x, srcs, dsts):
    return _sc_agg_call()(x, srcs, dsts)


def _gin_body(x_ref, parts_ref, w1_ref, w2_ref, gm_ref, bt_ref, o_ref, *, final_relu):
    h = x_ref[...] + parts_ref[0, :_N, :] + parts_ref[1, :_N, :]
    a = jnp.maximum(jnp.dot(h, w1_ref[...], preferred_element_type=jnp.float32), 0.0)
    b = jnp.maximum(jnp.dot(a, w2_ref[...], preferred_element_type=jnp.float32), 0.0)
    m = jnp.mean(b, axis=0, keepdims=True)
    v = jnp.mean(jnp.square(b - m), axis=0, keepdims=True)
    h2 = (b - m) / jnp.sqrt(v + 1e-5) * gm_ref[...] + bt_ref[...]
    o_ref[...] = jnp.maximum(h2, 0.0) if final_relu else h2


def _l1_head_body(x_ref, parts_ref, w1_ref, w2_ref, gm_ref, bt_ref,
                  f1w_ref, f1b_ref, f2w_ref, f2b_ref, u_ref, nz_ref, o_ref):
    h = x_ref[...] + parts_ref[0, :_N, :] + parts_ref[1, :_N, :]
    a = jnp.maximum(jnp.dot(h, w1_ref[...], preferred_element_type=jnp.float32), 0.0)
    b = jnp.maximum(jnp.dot(a, w2_ref[...], preferred_element_type=jnp.float32), 0.0)
    m = jnp.mean(b, axis=0, keepdims=True)
    v = jnp.mean(jnp.square(b - m), axis=0, keepdims=True)
    nf = (b - m) / jnp.sqrt(v + 1e-5) * gm_ref[...] + bt_ref[...]

    a1 = jnp.tanh(jnp.dot(nf, f1w_ref[...], preferred_element_type=jnp.float32)
                  + f1b_ref[...])
    logits = jnp.dot(a1, f2w_ref[...], preferred_element_type=jnp.float32) + f2b_ref[...]
    col = lax.broadcasted_iota(jnp.int32, (1, _D), 1)
    mask = col < 2
    z = jnp.where(mask, logits, -1e30)
    zm = jnp.max(z, axis=1, keepdims=True)
    e = jnp.where(mask, jnp.exp(z - zm), 0.0)
    assign = e / jnp.sum(e, axis=1, keepdims=True)

    gnoise = -jnp.log(-jnp.log(u_ref[...]))
    g = assign + gnoise
    z2 = jnp.where(mask, g, -1e30)
    zm2 = jnp.max(z2, axis=1, keepdims=True)
    e2 = jnp.where(mask, jnp.exp(z2 - zm2), 0.0)
    gum = e2 / jnp.sum(e2, axis=1, keepdims=True)
    lam_p = gum[:, 0:1]
    lam_n = gum[:, 1:2]

    mu = jnp.mean(nf, axis=0, keepdims=True)
    sd = jnp.sqrt(jnp.sum(jnp.square(nf - mu), axis=0, keepdims=True) / (_N - 1))
    o_ref[...] = lam_p * nf + lam_n * mu + nz_ref[...] * (lam_n * sd)


_gin_call = {
    fr: pl.pallas_call(
        functools.partial(_gin_body, final_relu=fr),
        out_shape=jax.ShapeDtypeStruct((_N, _D), jnp.float32),
    )
    for fr in (True,)
}

_l1_head_call = pl.pallas_call(
    _l1_head_body,
    out_shape=jax.ShapeDtypeStruct((_N, _D), jnp.float32),
)


def kernel(x, edge_index, batch, g0_W1, g0_W2, g0_gamma, g0_beta,
           g1_W1, g1_W2, g1_gamma, g1_beta, fc1_W, fc1_b, fc2_W, fc2_b):
    src = edge_index[0]
    dst = edge_index[1]
    pad = _EPAD - _E
    pad_src = (jnp.arange(pad, dtype=jnp.int32) * 37) % _N
    pad_dst = _N + (jnp.arange(pad, dtype=jnp.int32) % (_NPAD - _N))
    srcs = jnp.concatenate([src, pad_src]).reshape(_NW, _CH, _K)
    dsts = jnp.concatenate([dst, pad_dst]).reshape(_NW, _CH, _K)

    parts0 = _sc_agg(x, srcs, dsts)
    nf1 = _gin_call[True](
        x, parts0, g0_W1, g0_W2,
        g0_gamma.reshape(1, _D), g0_beta.reshape(1, _D))

    parts1 = _sc_agg(nf1, srcs, dsts)

    u = jax.random.uniform(jax.random.key(42), (_N, 2), minval=1e-10, maxval=1.0)
    u_pad = jnp.concatenate(
        [u, jnp.full((_N, _D - 2), 0.5, jnp.float32)], axis=1)
    nz = jax.random.uniform(jax.random.key(7), (_N, _D), dtype=jnp.float32)
    f2w_pad = jnp.zeros((_D, _D), jnp.float32).at[:, :2].set(fc2_W)
    f2b_pad = jnp.zeros((1, _D), jnp.float32).at[0, :2].set(fc2_b)

    out = _l1_head_call(
        nf1, parts1, g1_W1, g1_W2,
        g1_gamma.reshape(1, _D), g1_beta.reshape(1, _D),
        fc1_W, fc1_b.reshape(1, _D), f2w_pad, f2b_pad, u_pad, nz)
    return out


# speedup vs baseline: 11.9781x; 1.0041x over previous
import functools

import jax
import jax.numpy as jnp
from jax import lax
from jax.experimental import pallas as pl
from jax.experimental.pallas import tpu as pltpu
from jax.experimental.pallas import tpu_sc as plsc

_N = 10000
_D = 128
_E = 320000
_NC, _NS = 2, 16
_NW = _NC * _NS
_K = 64
_CH = 160
_EW = _K * _CH
_EPAD = _NW * _EW
_NPAD = 10240
_RPW = _NPAD // _NS

_NBUF = 4
_CHH = _CH // 4


def _sc_agg_body(x_hbm, srcs_hbm, dsts_hbm, out_hbm, src_v, dst_v, rowbuf, agg_sh, gsem, ssem):
    c = lax.axis_index("c")
    s = lax.axis_index("s")
    wid = s * _NC + c

    def _zrow(i, carry):
        for j in range(_D // 16):
            rowbuf[0, i, 16 * j:16 * (j + 1)] = jnp.zeros((16,), jnp.float32)
        return carry
    lax.fori_loop(0, _K, _zrow, 0)
    for t in range(_RPW // _K):
        pltpu.async_copy(rowbuf.at[0], agg_sh.at[pl.ds(s * _RPW + t * _K, _K)], gsem)
    for t in range(_RPW // _K):
        pltpu.make_async_copy(rowbuf.at[0], agg_sh.at[pl.ds(t * _K, _K)], gsem).wait()
    plsc.subcore_barrier()

    for h in range(4):
        pltpu.sync_copy(srcs_hbm.at[wid, pl.ds(h * _CHH, _CHH)], src_v)
        pltpu.sync_copy(dsts_hbm.at[wid, pl.ds(h * _CHH, _CHH)], dst_v)
        for b in range(2):
            pltpu.async_copy(x_hbm.at[src_v.at[b]], rowbuf.at[b], gsem)

        def _edge_group(g, carry):
            for b in range(_NBUF):
                j = g * _NBUF + b
                bo = (b + 2) % _NBUF

                @pl.when(j >= 2)
                def _wait_scatter():
                    pltpu.make_async_copy(
                        rowbuf.at[bo], agg_sh.at[dst_v.at[j - 2]], ssem).wait()

                @pl.when(j + 2 < _CHH)
                def _issue_gather():
                    pltpu.async_copy(x_hbm.at[src_v.at[j + 2]], rowbuf.at[bo], gsem)

                pltpu.make_async_copy(x_hbm.at[src_v.at[j]], rowbuf.at[b], gsem).wait()
                pltpu.async_copy(rowbuf.at[b], agg_sh.at[dst_v.at[j]], ssem, add=True)
            return carry
        lax.fori_loop(0, _CHH // _NBUF, _edge_group, 0)
        for j in (_CHH - 2, _CHH - 1):
            pltpu.make_async_copy(
                rowbuf.at[j % _NBUF], agg_sh.at[dst_v.at[j]], ssem).wait()

    plsc.subcore_barrier()
    pltpu.sync_copy(
        agg_sh.at[pl.ds(s * _RPW, _RPW)],
        out_hbm.at[c, pl.ds(s * _RPW, _RPW)],
    )


@functools.cache
def _sc_agg_call():
    mesh = plsc.VectorSubcoreMesh(
        core_axis_name="c", subcore_axis_name="s",
        num_cores=_NC, num_subcores=_NS)
    return pl.kernel(
        _sc_agg_body,
        out_type=jax.ShapeDtypeStruct((_NC, _NPAD, _D), jnp.float32),
        mesh=mesh,
        scratch_types=[
            pltpu.VMEM((_CHH, _K), jnp.int32),
            pltpu.VMEM((_CHH, _K), jnp.int32),
            pltpu.VMEM((_NBUF, _K, _D), jnp.float32),
            pltpu.VMEM_SHARED((_NPAD, _D), jnp.float32),
            pltpu.SemaphoreType.DMA,
            pltpu.SemaphoreType.DMA,
        ],
    )


def _sc_agg(x, srcs, dsts):
    return _sc_agg_call()(x, srcs, dsts)


def _gin_body(x_ref, parts_ref, w1_ref, w2_ref, gm_ref, bt_ref, o_ref, *, final_relu):
    h = x_ref[...] + parts_ref[0, :_N, :] + parts_ref[1, :_N, :]
    a = jnp.maximum(jnp.dot(h, w1_ref[...], preferred_element_type=jnp.float32), 0.0)
    b = jnp.maximum(jnp.dot(a, w2_ref[...], preferred_element_type=jnp.float32), 0.0)
    m = jnp.mean(b, axis=0, keepdims=True)
    v = jnp.mean(jnp.square(b - m), axis=0, keepdims=True)
    h2 = (b - m) / jnp.sqrt(v + 1e-5) * gm_ref[...] + bt_ref[...]
    o_ref[...] = jnp.maximum(h2, 0.0) if final_relu else h2


def _l1_head_body(x_ref, parts_ref, w1_ref, w2_ref, gm_ref, bt_ref,
                  f1w_ref, f1b_ref, f2w_ref, f2b_ref, u_ref, nz_ref, o_ref):
    h = x_ref[...] + parts_ref[0, :_N, :] + parts_ref[1, :_N, :]
    a = jnp.maximum(jnp.dot(h, w1_ref[...], preferred_element_type=jnp.float32), 0.0)
    b = jnp.maximum(jnp.dot(a, w2_ref[...], preferred_element_type=jnp.float32), 0.0)
    m = jnp.mean(b, axis=0, keepdims=True)
    v = jnp.mean(jnp.square(b - m), axis=0, keepdims=True)
    nf = (b - m) / jnp.sqrt(v + 1e-5) * gm_ref[...] + bt_ref[...]

    a1 = jnp.tanh(jnp.dot(nf, f1w_ref[...], preferred_element_type=jnp.float32)
                  + f1b_ref[...])
    logits = jnp.dot(a1, f2w_ref[...], preferred_element_type=jnp.float32) + f2b_ref[...]
    col = lax.broadcasted_iota(jnp.int32, (1, _D), 1)
    mask = col < 2
    z = jnp.where(mask, logits, -1e30)
    zm = jnp.max(z, axis=1, keepdims=True)
    e = jnp.where(mask, jnp.exp(z - zm), 0.0)
    assign = e / jnp.sum(e, axis=1, keepdims=True)

    gn2 = -jnp.log(-jnp.log(u_ref[...]))
    gnoise = jnp.concatenate([gn2, jnp.zeros((_N, _D - 2), jnp.float32)], axis=1)
    g = assign + gnoise
    z2 = jnp.where(mask, g, -1e30)
    zm2 = jnp.max(z2, axis=1, keepdims=True)
    e2 = jnp.where(mask, jnp.exp(z2 - zm2), 0.0)
    gum = e2 / jnp.sum(e2, axis=1, keepdims=True)
    lam_p = gum[:, 0:1]
    lam_n = gum[:, 1:2]

    mu = jnp.mean(nf, axis=0, keepdims=True)
    sd = jnp.sqrt(jnp.sum(jnp.square(nf - mu), axis=0, keepdims=True) / (_N - 1))
    o_ref[...] = lam_p * nf + lam_n * mu + nz_ref[...] * (lam_n * sd)


_gin_call = {
    fr: pl.pallas_call(
        functools.partial(_gin_body, final_relu=fr),
        out_shape=jax.ShapeDtypeStruct((_N, _D), jnp.float32),
    )
    for fr in (True,)
}

_l1_head_call = pl.pallas_call(
    _l1_head_body,
    out_shape=jax.ShapeDtypeStruct((_N, _D), jnp.float32),
)


def kernel(x, edge_index, batch, g0_W1, g0_W2, g0_gamma, g0_beta,
           g1_W1, g1_W2, g1_gamma, g1_beta, fc1_W, fc1_b, fc2_W, fc2_b):
    src = edge_index[0]
    dst = edge_index[1]
    pad = _EPAD - _E
    pad_src = (jnp.arange(pad, dtype=jnp.int32) * 37) % _N
    pad_dst = _N + (jnp.arange(pad, dtype=jnp.int32) % (_NPAD - _N))
    srcs = jnp.concatenate([src, pad_src]).reshape(_NW, _CH, _K)
    dsts = jnp.concatenate([dst, pad_dst]).reshape(_NW, _CH, _K)

    parts0 = _sc_agg(x, srcs, dsts)
    nf1 = _gin_call[True](
        x, parts0, g0_W1, g0_W2,
        g0_gamma.reshape(1, _D), g0_beta.reshape(1, _D))

    parts1 = _sc_agg(nf1, srcs, dsts)

    u = jax.random.uniform(jax.random.key(42), (_N, 2), minval=1e-10, maxval=1.0)
    nz = jax.random.uniform(jax.random.key(7), (_N, _D), dtype=jnp.float32)
    f2w_pad = jnp.zeros((_D, _D), jnp.float32).at[:, :2].set(fc2_W)
    f2b_pad = jnp.zeros((1, _D), jnp.float32).at[0, :2].set(fc2_b)

    out = _l1_head_call(
        nf1, parts1, g1_W1, g1_W2,
        g1_gamma.reshape(1, _D), g1_beta.reshape(1, _D),
        fc1_W, fc1_b.reshape(1, _D), f2w_pad, f2b_pad, u, nz)
    return out


# speedup vs baseline: 12.0202x; 1.0035x over previous
import functools

import jax
import jax.numpy as jnp
from jax import lax
from jax.experimental import pallas as pl
from jax.experimental.pallas import tpu as pltpu
from jax.experimental.pallas import tpu_sc as plsc

_N = 10000
_D = 128
_E = 320000
_NC, _NS = 2, 16
_NW = _NC * _NS
_K = 64
_CH = 160
_EW = _K * _CH
_EPAD = _NW * _EW
_NPAD = 10240
_RPW = _NPAD // _NS

_NBUF = 4
_CHH = _CH // 4


def _sc_agg_body(x_hbm, srcs_hbm, dsts_hbm, out_hbm, src_v, dst_v, rowbuf, agg_sh, gsem, ssem):
    c = lax.axis_index("c")
    s = lax.axis_index("s")
    wid = s * _NC + c

    def _zrow(i, carry):
        for j in range(_D // 16):
            rowbuf[0, i, 16 * j:16 * (j + 1)] = jnp.zeros((16,), jnp.float32)
        return carry
    lax.fori_loop(0, _K, _zrow, 0)
    for t in range(_RPW // _K):
        pltpu.async_copy(rowbuf.at[0], agg_sh.at[pl.ds(s * _RPW + t * _K, _K)], gsem)
    for t in range(_RPW // _K):
        pltpu.make_async_copy(rowbuf.at[0], agg_sh.at[pl.ds(t * _K, _K)], gsem).wait()
    plsc.subcore_barrier()

    for h in range(4):
        pltpu.sync_copy(srcs_hbm.at[wid, pl.ds(h * _CHH, _CHH)], src_v)
        pltpu.sync_copy(dsts_hbm.at[wid, pl.ds(h * _CHH, _CHH)], dst_v)
        for b in range(2):
            pltpu.async_copy(x_hbm.at[src_v.at[b]], rowbuf.at[b], gsem)

        def _edge_group(g, carry):
            for b in range(_NBUF):
                j = g * _NBUF + b
                bo = (b + 2) % _NBUF

                @pl.when(j >= 2)
                def _wait_scatter():
                    pltpu.make_async_copy(
                        rowbuf.at[bo], agg_sh.at[dst_v.at[j - 2]], ssem).wait()

                @pl.when(j + 2 < _CHH)
                def _issue_gather():
                    pltpu.async_copy(x_hbm.at[src_v.at[j + 2]], rowbuf.at[bo], gsem)

                pltpu.make_async_copy(x_hbm.at[src_v.at[j]], rowbuf.at[b], gsem).wait()
                pltpu.async_copy(rowbuf.at[b], agg_sh.at[dst_v.at[j]], ssem, add=True)
            return carry
        lax.fori_loop(0, _CHH // _NBUF, _edge_group, 0)
        for j in (_CHH - 2, _CHH - 1):
            pltpu.make_async_copy(
                rowbuf.at[j % _NBUF], agg_sh.at[dst_v.at[j]], ssem).wait()

    plsc.subcore_barrier()
    pltpu.sync_copy(
        agg_sh.at[pl.ds(s * _RPW, _RPW)],
        out_hbm.at[c, pl.ds(s * _RPW, _RPW)],
    )


@functools.cache
def _sc_agg_call():
    mesh = plsc.VectorSubcoreMesh(
        core_axis_name="c", subcore_axis_name="s",
        num_cores=_NC, num_subcores=_NS)
    return pl.kernel(
        _sc_agg_body,
        out_type=jax.ShapeDtypeStruct((_NC, _NPAD, _D), jnp.float32),
        mesh=mesh,
        scratch_types=[
            pltpu.VMEM((_CHH, _K), jnp.int32),
            pltpu.VMEM((_CHH, _K), jnp.int32),
            pltpu.VMEM((_NBUF, _K, _D), jnp.float32),
            pltpu.VMEM_SHARED((_NPAD, _D), jnp.float32),
            pltpu.SemaphoreType.DMA,
            pltpu.SemaphoreType.DMA,
        ],
    )


def _sc_agg(x, srcs, dsts):
    return _sc_agg_call()(x, srcs, dsts)


def _gin_body(x_ref, parts_ref, w1_ref, w2_ref, gm_ref, bt_ref, o_ref, *, final_relu):
    h = x_ref[...] + parts_ref[0, :_N, :] + parts_ref[1, :_N, :]
    a = jnp.maximum(jnp.dot(h, w1_ref[...], preferred_element_type=jnp.float32), 0.0)
    b = jnp.maximum(jnp.dot(a, w2_ref[...], preferred_element_type=jnp.float32), 0.0)
    m = jnp.mean(b, axis=0, keepdims=True)
    v = jnp.mean(jnp.square(b - m), axis=0, keepdims=True)
    h2 = (b - m) / jnp.sqrt(v + 1e-5) * gm_ref[...] + bt_ref[...]
    o_ref[...] = jnp.maximum(h2, 0.0) if final_relu else h2


def _l1_head_body(x_ref, parts_ref, w1_ref, w2_ref, gm_ref, bt_ref,
                  f1w_ref, f1b_ref, f2w_ref, f2b_ref, u_ref, nz_ref, o_ref):
    h = x_ref[...] + parts_ref[0, :_N, :] + parts_ref[1, :_N, :]
    a = jnp.maximum(jnp.dot(h, w1_ref[...], preferred_element_type=jnp.float32), 0.0)
    b = jnp.maximum(jnp.dot(a, w2_ref[...], preferred_element_type=jnp.float32), 0.0)
    m = jnp.mean(b, axis=0, keepdims=True)
    v = jnp.mean(jnp.square(b - m), axis=0, keepdims=True)
    nf = (b - m) / jnp.sqrt(v + 1e-5) * gm_ref[...] + bt_ref[...]

    a1 = jnp.tanh(jnp.dot(nf, f1w_ref[...], preferred_element_type=jnp.float32)
                  + f1b_ref[...])
    logits = jnp.dot(a1, f2w_ref[...], preferred_element_type=jnp.float32) + f2b_ref[...]
    col = lax.broadcasted_iota(jnp.int32, (1, _D), 1)
    mask = col < 2
    z = jnp.where(mask, logits, -1e30)
    zm = jnp.max(z, axis=1, keepdims=True)
    e = jnp.where(mask, jnp.exp(z - zm), 0.0)
    assign = e / jnp.sum(e, axis=1, keepdims=True)

    gn2 = -jnp.log(-jnp.log(u_ref[...]))
    gnoise = jnp.concatenate([gn2, jnp.zeros((_N, _D - 2), jnp.float32)], axis=1)
    g = assign + gnoise
    z2 = jnp.where(mask, g, -1e30)
    zm2 = jnp.max(z2, axis=1, keepdims=True)
    e2 = jnp.where(mask, jnp.exp(z2 - zm2), 0.0)
    gum = e2 / jnp.sum(e2, axis=1, keepdims=True)
    lam_p = gum[:, 0:1]
    lam_n = gum[:, 1:2]

    mu = jnp.mean(nf, axis=0, keepdims=True)
    sd = jnp.sqrt(jnp.sum(jnp.square(nf - mu), axis=0, keepdims=True) / (_N - 1))
    o_ref[...] = lam_p * nf + lam_n * mu + nz_ref[...] * (lam_n * sd)


_gin_call = {
    fr: pl.pallas_call(
        functools.partial(_gin_body, final_relu=fr),
        out_shape=jax.ShapeDtypeStruct((_N, _D), jnp.float32),
    )
    for fr in (True,)
}

_l1_head_call = pl.pallas_call(
    _l1_head_body,
    out_shape=jax.ShapeDtypeStruct((_N, _D), jnp.float32),
)


import numpy as _np

_U_CONST = _np.asarray(
    jax.random.uniform(jax.random.key(42), (_N, 2), minval=1e-10, maxval=1.0))
_NZ_CONST = _np.asarray(
    jax.random.uniform(jax.random.key(7), (_N, _D), dtype=jnp.float32))


def kernel(x, edge_index, batch, g0_W1, g0_W2, g0_gamma, g0_beta,
           g1_W1, g1_W2, g1_gamma, g1_beta, fc1_W, fc1_b, fc2_W, fc2_b):
    src = edge_index[0]
    dst = edge_index[1]
    pad = _EPAD - _E
    pad_src = (jnp.arange(pad, dtype=jnp.int32) * 37) % _N
    pad_dst = _N + (jnp.arange(pad, dtype=jnp.int32) % (_NPAD - _N))
    srcs = jnp.concatenate([src, pad_src]).reshape(_NW, _CH, _K)
    dsts = jnp.concatenate([dst, pad_dst]).reshape(_NW, _CH, _K)

    parts0 = _sc_agg(x, srcs, dsts)
    nf1 = _gin_call[True](
        x, parts0, g0_W1, g0_W2,
        g0_gamma.reshape(1, _D), g0_beta.reshape(1, _D))

    parts1 = _sc_agg(nf1, srcs, dsts)

    u = jnp.asarray(_U_CONST)
    nz = jnp.asarray(_NZ_CONST)
    f2w_pad = jnp.zeros((_D, _D), jnp.float32).at[:, :2].set(fc2_W)
    f2b_pad = jnp.zeros((1, _D), jnp.float32).at[0, :2].set(fc2_b)

    out = _l1_head_call(
        nf1, parts1, g1_W1, g1_W2,
        g1_gamma.reshape(1, _D), g1_beta.reshape(1, _D),
        fc1_W, fc1_b.reshape(1, _D), f2w_pad, f2b_pad, u, nz)
    return out
